# Initial kernel scaffold; baseline (speedup 1.0000x reference)
import jax
import jax.numpy as jnp
from jax.experimental import pallas as pl


def kernel(x, n1s, n1b, n2s, n2b, n3s, n3b, Wqkv, bqkv, Wo, bo, Wr, br, eln_s, eln_b, eW1, eb1, eW2, eb2):
    raise NotImplementedError("write your pallas kernel here")


# speedup vs baseline: 2.5321x; 2.5321x over previous
import jax
import jax.numpy as jnp
from jax.experimental import pallas as pl
from jax.experimental.pallas import tpu as pltpu

S, D, H, E = 2048, 1024, 16, 8
DH = D // H
F = 4 * D
EPS = 1e-05

QB = 512
TB = 512
FB = 1024


def _lnorm(x, s, b):
    m = jnp.mean(x, axis=-1, keepdims=True)
    v = jnp.mean((x - m) ** 2, axis=-1, keepdims=True)
    return (x - m) * jax.lax.rsqrt(v + EPS) * s + b


def _qkv_kernel(x_ref, s_ref, b_ref, w_ref, bias_ref, o_ref):
    xln = _lnorm(x_ref[...], s_ref[...], b_ref[...])
    o_ref[...] = jax.lax.dot_general(
        xln, w_ref[...], (((1,), (1,)), ((), ())),
        preferred_element_type=jnp.float32) + bias_ref[...]


def _attn_kernel(q_ref, k_ref, v_ref, o_ref):
    q = q_ref[0]
    sc = jax.lax.dot_general(
        q, k_ref[0], (((1,), (1,)), ((), ())),
        preferred_element_type=jnp.float32) * 0.125
    m = jnp.max(sc, axis=-1, keepdims=True)
    p = jnp.exp(sc - m)
    den = jnp.sum(p, axis=-1, keepdims=True)
    o = jnp.dot(p, v_ref[0], preferred_element_type=jnp.float32)
    o_ref[...] = (o / den)[None]


def _post_kernel(x_ref, ao_ref, wo_ref, bo_ref, s2_ref, b2_ref, wr_ref, br_ref,
                 x2_ref, z3_ref, lg_ref):
    attn = jax.lax.dot_general(
        ao_ref[...], wo_ref[...], (((1,), (1,)), ((), ())),
        preferred_element_type=jnp.float32) + bo_ref[...]
    x2 = x_ref[...] + attn
    x2_ref[...] = x2
    tok = _lnorm(x2, s2_ref[...], b2_ref[...])
    m = jnp.mean(tok, axis=-1, keepdims=True)
    v = jnp.mean((tok - m) ** 2, axis=-1, keepdims=True)
    z3_ref[...] = (tok - m) * jax.lax.rsqrt(v + EPS)
    lg_ref[...] = jax.lax.dot_general(
        tok, wr_ref[...], (((1,), (1,)), ((), ())),
        preferred_element_type=jnp.float32) + br_ref[...]


def _expert_kernel(z3_ref, es_ref, eb_ref, w1_ref, b1_ref, w2_ref, b2_ref,
                   eo_ref):
    fb = pl.program_id(1)
    hl = (z3_ref[...] * es_ref[0] + eb_ref[0]).astype(jnp.bfloat16)
    w1 = w1_ref[0].astype(jnp.bfloat16)
    hh = jax.lax.dot_general(
        hl, w1, (((1,), (1,)), ((), ())),
        preferred_element_type=jnp.float32) + b1_ref[0]
    g = (hh * 0.5 * (1.0 + jax.lax.erf(hh * 0.7071067811865476))).astype(
        jnp.bfloat16)
    w2 = w2_ref[0].astype(jnp.bfloat16)
    part = jax.lax.dot_general(
        g, w2, (((1,), (1,)), ((), ())),
        preferred_element_type=jnp.float32)

    @pl.when(fb == 0)
    def _():
        eo_ref[...] = (part + b2_ref[0])[None]

    @pl.when(fb != 0)
    def _():
        eo_ref[...] = eo_ref[...] + part[None]


def _combine_kernel(lg_ref, eo_ref, x2_ref, s3_ref, b3_ref, o_ref):
    lg = lg_ref[...]
    iota = jax.lax.broadcasted_iota(jnp.int32, lg.shape, 1)
    m1 = jnp.max(lg, axis=-1, keepdims=True)
    i1 = jnp.min(jnp.where(lg == m1, iota, E), axis=-1, keepdims=True)
    lg2 = jnp.where(iota == i1, -jnp.inf, lg)
    m2 = jnp.max(lg2, axis=-1, keepdims=True)
    i2 = jnp.min(jnp.where(lg2 == m2, iota, E), axis=-1, keepdims=True)
    w1 = 1.0 / (1.0 + jnp.exp(m2 - m1))
    w2 = 1.0 - w1
    wfull = jnp.where(iota == i1, w1, 0.0) + jnp.where(iota == i2, w2, 0.0)
    eo = eo_ref[...]
    wt = wfull.T[:, :, None]
    comb = jnp.sum(wt * eo, axis=0)
    x3 = x2_ref[...] + comb
    o_ref[...] = _lnorm(x3, s3_ref[...], b3_ref[...])


def kernel(x, n1s, n1b, n2s, n2b, n3s, n3b, Wqkv, bqkv, Wo, bo, Wr, br,
           eln_s, eln_b, eW1, eb1, eW2, eb2):
    f32 = jnp.float32
    x2d = x.reshape(S, D)
    r1 = lambda a: a.reshape(1, -1)

    qkv = pl.pallas_call(
        _qkv_kernel,
        grid=(6,),
        in_specs=[
            pl.BlockSpec((S, D), lambda j: (0, 0)),
            pl.BlockSpec((1, D), lambda j: (0, 0)),
            pl.BlockSpec((1, D), lambda j: (0, 0)),
            pl.BlockSpec((512, D), lambda j: (j, 0)),
            pl.BlockSpec((1, 512), lambda j: (0, j)),
        ],
        out_specs=pl.BlockSpec((S, 512), lambda j: (0, j)),
        out_shape=jax.ShapeDtypeStruct((S, 3 * D), f32),
    )(x2d, r1(n1s), r1(n1b), Wqkv, r1(bqkv))

    qkvh = qkv.reshape(S, 3 * H, DH).transpose(1, 0, 2)
    aoh = pl.pallas_call(
        _attn_kernel,
        grid=(H, S // QB),
        in_specs=[
            pl.BlockSpec((1, QB, DH), lambda h, qb: (h, qb, 0)),
            pl.BlockSpec((1, S, DH), lambda h, qb: (H + h, 0, 0)),
            pl.BlockSpec((1, S, DH), lambda h, qb: (2 * H + h, 0, 0)),
        ],
        out_specs=pl.BlockSpec((1, QB, DH), lambda h, qb: (h, qb, 0)),
        out_shape=jax.ShapeDtypeStruct((H, S, DH), f32),
    )(qkvh, qkvh, qkvh)
    ao = aoh.transpose(1, 0, 2).reshape(S, D)

    x2, z3, lg = pl.pallas_call(
        _post_kernel,
        grid=(S // TB,),
        in_specs=[
            pl.BlockSpec((TB, D), lambda t: (t, 0)),
            pl.BlockSpec((TB, D), lambda t: (t, 0)),
            pl.BlockSpec((D, D), lambda t: (0, 0)),
            pl.BlockSpec((1, D), lambda t: (0, 0)),
            pl.BlockSpec((1, D), lambda t: (0, 0)),
            pl.BlockSpec((1, D), lambda t: (0, 0)),
            pl.BlockSpec((E, D), lambda t: (0, 0)),
            pl.BlockSpec((1, E), lambda t: (0, 0)),
        ],
        out_specs=[
            pl.BlockSpec((TB, D), lambda t: (t, 0)),
            pl.BlockSpec((TB, D), lambda t: (t, 0)),
            pl.BlockSpec((TB, E), lambda t: (t, 0)),
        ],
        out_shape=[
            jax.ShapeDtypeStruct((S, D), f32),
            jax.ShapeDtypeStruct((S, D), f32),
            jax.ShapeDtypeStruct((S, E), f32),
        ],
    )(x2d, ao, Wo, r1(bo), r1(n2s), r1(n2b), Wr, r1(br))

    eo = pl.pallas_call(
        _expert_kernel,
        grid=(E, F // FB),
        in_specs=[
            pl.BlockSpec((S, D), lambda e, fb: (0, 0)),
            pl.BlockSpec((1, 1, D), lambda e, fb: (e, 0, 0)),
            pl.BlockSpec((1, 1, D), lambda e, fb: (e, 0, 0)),
            pl.BlockSpec((1, FB, D), lambda e, fb: (e, fb, 0)),
            pl.BlockSpec((1, 1, FB), lambda e, fb: (e, 0, fb)),
            pl.BlockSpec((1, D, FB), lambda e, fb: (e, 0, fb)),
            pl.BlockSpec((1, 1, D), lambda e, fb: (e, 0, 0)),
        ],
        out_specs=pl.BlockSpec((1, S, D), lambda e, fb: (e, 0, 0)),
        out_shape=jax.ShapeDtypeStruct((E, S, D), f32),
    )(z3, eln_s[:, None, :], eln_b[:, None, :], eW1, eb1[:, None, :],
      eW2, eb2[:, None, :])

    out = pl.pallas_call(
        _combine_kernel,
        grid=(S // TB,),
        in_specs=[
            pl.BlockSpec((TB, E), lambda t: (t, 0)),
            pl.BlockSpec((E, TB, D), lambda t: (0, t, 0)),
            pl.BlockSpec((TB, D), lambda t: (t, 0)),
            pl.BlockSpec((1, D), lambda t: (0, 0)),
            pl.BlockSpec((1, D), lambda t: (0, 0)),
        ],
        out_specs=pl.BlockSpec((TB, D), lambda t: (t, 0)),
        out_shape=jax.ShapeDtypeStruct((S, D), f32),
    )(lg, eo, x2, r1(n3s), r1(n3b))

    return out.reshape(1, S, D)


# speedup vs baseline: 3.1484x; 1.2434x over previous
import jax
import jax.numpy as jnp
from jax.experimental import pallas as pl
from jax.experimental.pallas import tpu as pltpu

S, D, H, E = 2048, 1024, 16, 8
DH = D // H
F = 4 * D
EPS = 1e-05

QB = 512
TB = 512
FB = 1024
CAP = 2048
TBS = 512
NFB = F // FB
NJB = CAP // TBS
BIGSLOT = 1e6


def _lnorm(x, s, b):
    m = jnp.mean(x, axis=-1, keepdims=True)
    v = jnp.mean((x - m) ** 2, axis=-1, keepdims=True)
    return (x - m) * jax.lax.rsqrt(v + EPS) * s + b


def _qkv_kernel(x_ref, s_ref, b_ref, w_ref, bias_ref, o_ref):
    xln = _lnorm(x_ref[...], s_ref[...], b_ref[...])
    o_ref[...] = jax.lax.dot_general(
        xln, w_ref[...], (((1,), (1,)), ((), ())),
        preferred_element_type=jnp.float32) + bias_ref[...]


def _attn_kernel(q_ref, k_ref, v_ref, o_ref):
    outs = []
    for i in range(2):
        q = q_ref[:, i * DH:(i + 1) * DH]
        k = k_ref[:, i * DH:(i + 1) * DH]
        v = v_ref[:, i * DH:(i + 1) * DH]
        sc = jax.lax.dot_general(
            q, k, (((1,), (1,)), ((), ())),
            preferred_element_type=jnp.float32) * 0.125
        m = jnp.max(sc, axis=-1, keepdims=True)
        p = jnp.exp(sc - m)
        den = jnp.sum(p, axis=-1, keepdims=True)
        o = jnp.dot(p, v, preferred_element_type=jnp.float32)
        outs.append(o / den)
    o_ref[...] = jnp.concatenate(outs, axis=1)


def _post_kernel(x_ref, ao_ref, wo_ref, bo_ref, s2_ref, b2_ref, wr_ref,
                 br_ref, x2_ref, z3b_ref, s1_ref, s2o_ref, wt1_ref, wt2_ref,
                 cnt_ref, base_ref):
    tb = pl.program_id(0)

    @pl.when(tb == 0)
    def _():
        base_ref[...] = jnp.zeros_like(base_ref)

    attn = jax.lax.dot_general(
        ao_ref[...], wo_ref[...], (((1,), (1,)), ((), ())),
        preferred_element_type=jnp.float32) + bo_ref[...]
    x2 = x_ref[...] + attn
    x2_ref[...] = x2
    tok = _lnorm(x2, s2_ref[...], b2_ref[...])
    m = jnp.mean(tok, axis=-1, keepdims=True)
    v = jnp.mean((tok - m) ** 2, axis=-1, keepdims=True)
    z3b_ref[...] = ((tok - m) * jax.lax.rsqrt(v + EPS)).astype(jnp.bfloat16)
    lg = jax.lax.dot_general(
        tok, wr_ref[...], (((1,), (1,)), ((), ())),
        preferred_element_type=jnp.float32) + br_ref[...]

    iota = jax.lax.broadcasted_iota(jnp.int32, (TB, E), 1)
    m1 = jnp.max(lg, axis=-1, keepdims=True)
    i1 = jnp.min(jnp.where(lg == m1, iota, E), axis=-1, keepdims=True)
    lg2 = jnp.where(iota == i1, -jnp.inf, lg)
    m2 = jnp.max(lg2, axis=-1, keepdims=True)
    i2 = jnp.min(jnp.where(lg2 == m2, iota, E), axis=-1, keepdims=True)
    w1 = 1.0 / (1.0 + jnp.exp(m2 - m1))
    w2 = 1.0 - w1
    sel1 = iota == i1
    sel2 = iota == i2
    self_f = (sel1 | sel2).astype(jnp.float32)
    wt1_ref[...] = w1
    wt2_ref[...] = w2

    r_iota = jax.lax.broadcasted_iota(jnp.int32, (TB, TB), 0)
    c_iota = jax.lax.broadcasted_iota(jnp.int32, (TB, TB), 1)
    lt = (c_iota < r_iota).astype(jnp.float32)
    rank = jnp.floor(jnp.dot(lt, self_f,
                             preferred_element_type=jnp.float32) + 0.5)
    base = base_ref[...]
    e_cap = iota.astype(jnp.float32) * CAP
    gslot = e_cap + base + rank
    s1_ref[...] = jnp.sum(jnp.where(sel1, gslot, 0.0), axis=-1,
                          keepdims=True)
    s2o_ref[...] = jnp.sum(jnp.where(sel2, gslot, 0.0), axis=-1,
                           keepdims=True)
    base_ref[...] = base + jnp.sum(self_f, axis=0, keepdims=True)

    @pl.when(tb == pl.num_programs(0) - 1)
    def _():
        cnt_ref[...] = base_ref[...]


def _moe_kernel(cnt_ref, z3b_ref, s1_ref, s2_ref, wt1_ref, wt2_ref,
                es_ref, ebv_ref, w1_ref, b1_ref, w2_ref, b2_ref,
                out_ref, geo_ref, eo_ref):
    e = pl.program_id(0)
    fb = pl.program_id(1)
    jb = pl.program_id(2)

    @pl.when((e == 0) & (fb == 0) & (jb == 0))
    def _():
        out_ref[...] = jnp.zeros_like(out_ref)

    base = jb * TBS

    @pl.when(base < cnt_ref[e])
    def _work():
        off = (e * CAP + base).astype(jnp.float32)
        lane = jax.lax.broadcasted_iota(
            jnp.int32, (S, TBS), 1).astype(jnp.float32)
        m1 = jnp.abs(s1_ref[...] - off - lane) < 0.5
        m2 = jnp.abs(s2_ref[...] - off - lane) < 0.5
        pt = m1 | m2
        ptb = pt.astype(jnp.bfloat16)

        @pl.when(fb == 0)
        def _gather():
            g = jax.lax.dot_general(
                ptb, z3b_ref[...], (((0,), (0,)), ((), ())),
                preferred_element_type=jnp.float32)
            hl = g * es_ref[0] + ebv_ref[0]
            geo_ref[pl.ds(base, TBS), :] = hl.astype(jnp.bfloat16)

        hl = geo_ref[pl.ds(base, TBS), :]
        w1 = w1_ref[0].astype(jnp.bfloat16)
        hh = jax.lax.dot_general(
            hl, w1, (((1,), (1,)), ((), ())),
            preferred_element_type=jnp.float32) + b1_ref[0]
        gg = (hh * 0.5 * (1.0 + jax.lax.erf(hh * 0.7071067811865476))
              ).astype(jnp.bfloat16)
        w2 = w2_ref[0].astype(jnp.bfloat16)
        part = jax.lax.dot_general(
            gg, w2, (((1,), (1,)), ((), ())),
            preferred_element_type=jnp.float32)

        @pl.when(fb == 0)
        def _eo_init():
            eo_ref[pl.ds(base, TBS), :] = (part + b2_ref[0]).astype(
                jnp.bfloat16)

        @pl.when((fb > 0) & (fb < NFB - 1))
        def _eo_acc():
            eo_ref[pl.ds(base, TBS), :] = (
                eo_ref[pl.ds(base, TBS), :].astype(jnp.float32) + part
            ).astype(jnp.bfloat16)

        @pl.when(fb == NFB - 1)
        def _combine():
            eof = (eo_ref[pl.ds(base, TBS), :].astype(jnp.float32)
                   + part).astype(jnp.bfloat16)
            wpt = (jnp.where(m1, wt1_ref[...], 0.0)
                   + jnp.where(m2, wt2_ref[...], 0.0)).astype(jnp.bfloat16)
            out_ref[...] += jax.lax.dot_general(
                wpt, eof, (((1,), (0,)), ((), ())),
                preferred_element_type=jnp.float32)


def _final_kernel(x2_ref, comb_ref, s3_ref, b3_ref, o_ref):
    o_ref[...] = _lnorm(x2_ref[...] + comb_ref[...], s3_ref[...], b3_ref[...])


def kernel(x, n1s, n1b, n2s, n2b, n3s, n3b, Wqkv, bqkv, Wo, bo, Wr, br,
           eln_s, eln_b, eW1, eb1, eW2, eb2):
    f32 = jnp.float32
    x2d = x.reshape(S, D)
    r1 = lambda a: a.reshape(1, -1)

    qkv = pl.pallas_call(
        _qkv_kernel,
        grid=(6,),
        in_specs=[
            pl.BlockSpec((S, D), lambda j: (0, 0)),
            pl.BlockSpec((1, D), lambda j: (0, 0)),
            pl.BlockSpec((1, D), lambda j: (0, 0)),
            pl.BlockSpec((512, D), lambda j: (j, 0)),
            pl.BlockSpec((1, 512), lambda j: (0, j)),
        ],
        out_specs=pl.BlockSpec((S, 512), lambda j: (0, j)),
        out_shape=jax.ShapeDtypeStruct((S, 3 * D), f32),
    )(x2d, r1(n1s), r1(n1b), Wqkv, r1(bqkv))

    ao = pl.pallas_call(
        _attn_kernel,
        grid=(H // 2, S // QB),
        in_specs=[
            pl.BlockSpec((QB, 2 * DH), lambda hp, qb: (qb, hp)),
            pl.BlockSpec((S, 2 * DH), lambda hp, qb: (0, H // 2 + hp)),
            pl.BlockSpec((S, 2 * DH), lambda hp, qb: (0, H + hp)),
        ],
        out_specs=pl.BlockSpec((QB, 2 * DH), lambda hp, qb: (qb, hp)),
        out_shape=jax.ShapeDtypeStruct((S, D), f32),
    )(qkv, qkv, qkv)

    x2, z3b, slot1, slot2, wt1, wt2, counts = pl.pallas_call(
        _post_kernel,
        grid=(S // TB,),
        in_specs=[
            pl.BlockSpec((TB, D), lambda t: (t, 0)),
            pl.BlockSpec((TB, D), lambda t: (t, 0)),
            pl.BlockSpec((D, D), lambda t: (0, 0)),
            pl.BlockSpec((1, D), lambda t: (0, 0)),
            pl.BlockSpec((1, D), lambda t: (0, 0)),
            pl.BlockSpec((1, D), lambda t: (0, 0)),
            pl.BlockSpec((E, D), lambda t: (0, 0)),
            pl.BlockSpec((1, E), lambda t: (0, 0)),
        ],
        out_specs=[
            pl.BlockSpec((TB, D), lambda t: (t, 0)),
            pl.BlockSpec((TB, D), lambda t: (t, 0)),
            pl.BlockSpec((TB, 1), lambda t: (t, 0)),
            pl.BlockSpec((TB, 1), lambda t: (t, 0)),
            pl.BlockSpec((TB, 1), lambda t: (t, 0)),
            pl.BlockSpec((TB, 1), lambda t: (t, 0)),
            pl.BlockSpec((1, E), lambda t: (0, 0)),
        ],
        out_shape=[
            jax.ShapeDtypeStruct((S, D), f32),
            jax.ShapeDtypeStruct((S, D), jnp.bfloat16),
            jax.ShapeDtypeStruct((S, 1), f32),
            jax.ShapeDtypeStruct((S, 1), f32),
            jax.ShapeDtypeStruct((S, 1), f32),
            jax.ShapeDtypeStruct((S, 1), f32),
            jax.ShapeDtypeStruct((1, E), f32),
        ],
        scratch_shapes=[pltpu.VMEM((1, E), f32)],
    )(x2d, ao, Wo, r1(bo), r1(n2s), r1(n2b), Wr, r1(br))

    cnt_i = counts.reshape(E).astype(jnp.int32)

    comb = pl.pallas_call(
        _moe_kernel,
        grid_spec=pltpu.PrefetchScalarGridSpec(
            num_scalar_prefetch=1,
            grid=(E, NFB, NJB),
            in_specs=[
                pl.BlockSpec((S, D), lambda e, fb, jb, c: (0, 0)),
                pl.BlockSpec((S, 1), lambda e, fb, jb, c: (0, 0)),
                pl.BlockSpec((S, 1), lambda e, fb, jb, c: (0, 0)),
                pl.BlockSpec((S, 1), lambda e, fb, jb, c: (0, 0)),
                pl.BlockSpec((S, 1), lambda e, fb, jb, c: (0, 0)),
                pl.BlockSpec((1, 1, D), lambda e, fb, jb, c: (e, 0, 0)),
                pl.BlockSpec((1, 1, D), lambda e, fb, jb, c: (e, 0, 0)),
                pl.BlockSpec((1, FB, D), lambda e, fb, jb, c: (e, fb, 0)),
                pl.BlockSpec((1, 1, FB), lambda e, fb, jb, c: (e, 0, fb)),
                pl.BlockSpec((1, D, FB), lambda e, fb, jb, c: (e, 0, fb)),
                pl.BlockSpec((1, 1, D), lambda e, fb, jb, c: (e, 0, 0)),
            ],
            out_specs=pl.BlockSpec((S, D), lambda e, fb, jb, c: (0, 0)),
            scratch_shapes=[
                pltpu.VMEM((CAP, D), jnp.bfloat16),
                pltpu.VMEM((CAP, D), jnp.bfloat16),
            ],
        ),
        out_shape=jax.ShapeDtypeStruct((S, D), f32),
    )(cnt_i, z3b, slot1, slot2, wt1, wt2, eln_s[:, None, :],
      eln_b[:, None, :], eW1, eb1[:, None, :], eW2, eb2[:, None, :])

    out = pl.pallas_call(
        _final_kernel,
        grid=(S // TB,),
        in_specs=[
            pl.BlockSpec((TB, D), lambda t: (t, 0)),
            pl.BlockSpec((TB, D), lambda t: (t, 0)),
            pl.BlockSpec((1, D), lambda t: (0, 0)),
            pl.BlockSpec((1, D), lambda t: (0, 0)),
        ],
        out_specs=pl.BlockSpec((TB, D), lambda t: (t, 0)),
        out_shape=jax.ShapeDtypeStruct((S, D), f32),
    )(x2, comb, r1(n3s), r1(n3b))

    return out.reshape(1, S, D)


# speedup vs baseline: 3.3640x; 1.0685x over previous
import jax
import jax.numpy as jnp
from jax.experimental import pallas as pl
from jax.experimental.pallas import tpu as pltpu

S, D, H, E = 2048, 1024, 16, 8
DH = D // H
F = 4 * D
EPS = 1e-05

QB = 512
TB = 512
FB = 1024
CAP = 2048
TBS = 512
NFB = F // FB
NJB = CAP // TBS
BIGSLOT = 1e6


def _lnorm(x, s, b):
    m = jnp.mean(x, axis=-1, keepdims=True)
    v = jnp.mean((x - m) ** 2, axis=-1, keepdims=True)
    return (x - m) * jax.lax.rsqrt(v + EPS) * s + b


def _qkv_kernel(x_ref, s_ref, b_ref, w_ref, bias_ref, o_ref):
    xln = _lnorm(x_ref[...], s_ref[...], b_ref[...])
    o_ref[...] = jax.lax.dot_general(
        xln, w_ref[...], (((1,), (1,)), ((), ())),
        preferred_element_type=jnp.float32) + bias_ref[...]


def _attn_kernel(q_ref, k_ref, v_ref, o_ref):
    outs = []
    for i in range(2):
        q = q_ref[:, i * DH:(i + 1) * DH]
        k = k_ref[:, i * DH:(i + 1) * DH]
        v = v_ref[:, i * DH:(i + 1) * DH]
        sc = jax.lax.dot_general(
            q, k, (((1,), (1,)), ((), ())),
            preferred_element_type=jnp.float32) * 0.125
        m = jnp.max(sc, axis=-1, keepdims=True)
        p = jnp.exp(sc - m)
        den = jnp.sum(p, axis=-1, keepdims=True)
        o = jnp.dot(p, v, preferred_element_type=jnp.float32)
        outs.append(o / den)
    o_ref[...] = jnp.concatenate(outs, axis=1)


def _post_kernel(x_ref, ao_ref, wo_ref, bo_ref, s2_ref, b2_ref, wr_ref,
                 br_ref, x2_ref, z3b_ref, s1_ref, s2o_ref, wt1_ref, wt2_ref,
                 cnt_ref, base_ref):
    tb = pl.program_id(0)

    @pl.when(tb == 0)
    def _():
        base_ref[...] = jnp.zeros_like(base_ref)

    attn = jax.lax.dot_general(
        ao_ref[...], wo_ref[...], (((1,), (1,)), ((), ())),
        preferred_element_type=jnp.float32) + bo_ref[...]
    x2 = x_ref[...] + attn
    x2_ref[...] = x2
    tok = _lnorm(x2, s2_ref[...], b2_ref[...])
    m = jnp.mean(tok, axis=-1, keepdims=True)
    v = jnp.mean((tok - m) ** 2, axis=-1, keepdims=True)
    z3b_ref[...] = ((tok - m) * jax.lax.rsqrt(v + EPS)).astype(jnp.bfloat16)
    lg = jax.lax.dot_general(
        tok, wr_ref[...], (((1,), (1,)), ((), ())),
        preferred_element_type=jnp.float32) + br_ref[...]

    iota = jax.lax.broadcasted_iota(jnp.int32, (TB, E), 1)
    m1 = jnp.max(lg, axis=-1, keepdims=True)
    i1 = jnp.min(jnp.where(lg == m1, iota, E), axis=-1, keepdims=True)
    lg2 = jnp.where(iota == i1, -jnp.inf, lg)
    m2 = jnp.max(lg2, axis=-1, keepdims=True)
    i2 = jnp.min(jnp.where(lg2 == m2, iota, E), axis=-1, keepdims=True)
    w1 = 1.0 / (1.0 + jnp.exp(m2 - m1))
    w2 = 1.0 - w1
    sel1 = iota == i1
    sel2 = iota == i2
    self_f = (sel1 | sel2).astype(jnp.float32)
    wt1_ref[...] = w1
    wt2_ref[...] = w2

    r_iota = jax.lax.broadcasted_iota(jnp.int32, (TB, TB), 0)
    c_iota = jax.lax.broadcasted_iota(jnp.int32, (TB, TB), 1)
    lt = (c_iota < r_iota).astype(jnp.float32)
    rank = jnp.floor(jnp.dot(lt, self_f,
                             preferred_element_type=jnp.float32) + 0.5)
    base = base_ref[...]
    e_cap = iota.astype(jnp.float32) * CAP
    gslot = e_cap + base + rank
    s1_ref[...] = jnp.sum(jnp.where(sel1, gslot, 0.0), axis=-1,
                          keepdims=True)
    s2o_ref[...] = jnp.sum(jnp.where(sel2, gslot, 0.0), axis=-1,
                           keepdims=True)
    base_ref[...] = base + jnp.sum(self_f, axis=0, keepdims=True)

    @pl.when(tb == pl.num_programs(0) - 1)
    def _():
        cnt_ref[...] = base_ref[...]


def _moe_kernel(cnt_ref, z3b_ref, s1_ref, s2_ref, wt1_ref, wt2_ref,
                es_ref, ebv_ref, w1_ref, b1_ref, w2_ref, b2_ref,
                out_ref, geo_ref, eo_ref):
    e = pl.program_id(0)
    fb = pl.program_id(1)
    jb = pl.program_id(2)

    @pl.when((e == 0) & (fb == 0) & (jb == 0))
    def _():
        out_ref[...] = jnp.zeros_like(out_ref)

    base = jb * TBS

    @pl.when(base < cnt_ref[e])
    def _work():
        off = (e * CAP + base).astype(jnp.float32)

        def _masks():
            lane = jax.lax.broadcasted_iota(
                jnp.int32, (S, TBS), 1).astype(jnp.float32)
            m1 = jnp.abs(s1_ref[...] - off - lane) < 0.5
            m2 = jnp.abs(s2_ref[...] - off - lane) < 0.5
            return m1, m2

        @pl.when(fb == 0)
        def _gather():
            m1, m2 = _masks()
            ptb = (m1 | m2).astype(jnp.bfloat16)
            g = jax.lax.dot_general(
                ptb, z3b_ref[...], (((0,), (0,)), ((), ())),
                preferred_element_type=jnp.float32)
            hl = g * es_ref[0] + ebv_ref[0]
            geo_ref[pl.ds(base, TBS), :] = hl.astype(jnp.bfloat16)

        hl = geo_ref[pl.ds(base, TBS), :]
        w1 = w1_ref[0].astype(jnp.bfloat16)
        hh = jax.lax.dot_general(
            hl, w1, (((1,), (1,)), ((), ())),
            preferred_element_type=jnp.float32) + b1_ref[0]
        gg = (hh * 0.5 * (1.0 + jax.lax.erf(hh * 0.7071067811865476))
              ).astype(jnp.bfloat16)
        w2 = w2_ref[0].astype(jnp.bfloat16)
        part = jax.lax.dot_general(
            gg, w2, (((1,), (1,)), ((), ())),
            preferred_element_type=jnp.float32)

        @pl.when(fb == 0)
        def _eo_init():
            eo_ref[pl.ds(base, TBS), :] = (part + b2_ref[0]).astype(
                jnp.bfloat16)

        @pl.when((fb > 0) & (fb < NFB - 1))
        def _eo_acc():
            eo_ref[pl.ds(base, TBS), :] = (
                eo_ref[pl.ds(base, TBS), :].astype(jnp.float32) + part
            ).astype(jnp.bfloat16)

        @pl.when(fb == NFB - 1)
        def _combine():
            eof = (eo_ref[pl.ds(base, TBS), :].astype(jnp.float32)
                   + part).astype(jnp.bfloat16)
            m1, m2 = _masks()
            wpt = (jnp.where(m1, wt1_ref[...], 0.0)
                   + jnp.where(m2, wt2_ref[...], 0.0)).astype(jnp.bfloat16)
            out_ref[...] += jax.lax.dot_general(
                wpt, eof, (((1,), (0,)), ((), ())),
                preferred_element_type=jnp.float32)


def _final_kernel(x2_ref, comb_ref, s3_ref, b3_ref, o_ref):
    o_ref[...] = _lnorm(x2_ref[...] + comb_ref[...], s3_ref[...], b3_ref[...])


def kernel(x, n1s, n1b, n2s, n2b, n3s, n3b, Wqkv, bqkv, Wo, bo, Wr, br,
           eln_s, eln_b, eW1, eb1, eW2, eb2):
    f32 = jnp.float32
    x2d = x.reshape(S, D)
    r1 = lambda a: a.reshape(1, -1)

    qkv = pl.pallas_call(
        _qkv_kernel,
        grid=(6,),
        in_specs=[
            pl.BlockSpec((S, D), lambda j: (0, 0)),
            pl.BlockSpec((1, D), lambda j: (0, 0)),
            pl.BlockSpec((1, D), lambda j: (0, 0)),
            pl.BlockSpec((512, D), lambda j: (j, 0)),
            pl.BlockSpec((1, 512), lambda j: (0, j)),
        ],
        out_specs=pl.BlockSpec((S, 512), lambda j: (0, j)),
        out_shape=jax.ShapeDtypeStruct((S, 3 * D), f32),
    )(x2d, r1(n1s), r1(n1b), Wqkv, r1(bqkv))

    ao = pl.pallas_call(
        _attn_kernel,
        grid=(H // 2, S // QB),
        in_specs=[
            pl.BlockSpec((QB, 2 * DH), lambda hp, qb: (qb, hp)),
            pl.BlockSpec((S, 2 * DH), lambda hp, qb: (0, H // 2 + hp)),
            pl.BlockSpec((S, 2 * DH), lambda hp, qb: (0, H + hp)),
        ],
        out_specs=pl.BlockSpec((QB, 2 * DH), lambda hp, qb: (qb, hp)),
        out_shape=jax.ShapeDtypeStruct((S, D), f32),
    )(qkv, qkv, qkv)

    x2, z3b, slot1, slot2, wt1, wt2, counts = pl.pallas_call(
        _post_kernel,
        grid=(S // TB,),
        in_specs=[
            pl.BlockSpec((TB, D), lambda t: (t, 0)),
            pl.BlockSpec((TB, D), lambda t: (t, 0)),
            pl.BlockSpec((D, D), lambda t: (0, 0)),
            pl.BlockSpec((1, D), lambda t: (0, 0)),
            pl.BlockSpec((1, D), lambda t: (0, 0)),
            pl.BlockSpec((1, D), lambda t: (0, 0)),
            pl.BlockSpec((E, D), lambda t: (0, 0)),
            pl.BlockSpec((1, E), lambda t: (0, 0)),
        ],
        out_specs=[
            pl.BlockSpec((TB, D), lambda t: (t, 0)),
            pl.BlockSpec((TB, D), lambda t: (t, 0)),
            pl.BlockSpec((TB, 1), lambda t: (t, 0)),
            pl.BlockSpec((TB, 1), lambda t: (t, 0)),
            pl.BlockSpec((TB, 1), lambda t: (t, 0)),
            pl.BlockSpec((TB, 1), lambda t: (t, 0)),
            pl.BlockSpec((1, E), lambda t: (0, 0)),
        ],
        out_shape=[
            jax.ShapeDtypeStruct((S, D), f32),
            jax.ShapeDtypeStruct((S, D), jnp.bfloat16),
            jax.ShapeDtypeStruct((S, 1), f32),
            jax.ShapeDtypeStruct((S, 1), f32),
            jax.ShapeDtypeStruct((S, 1), f32),
            jax.ShapeDtypeStruct((S, 1), f32),
            jax.ShapeDtypeStruct((1, E), f32),
        ],
        scratch_shapes=[pltpu.VMEM((1, E), f32)],
    )(x2d, ao, Wo, r1(bo), r1(n2s), r1(n2b), Wr, r1(br))

    cnt_i = counts.reshape(E).astype(jnp.int32)

    comb = pl.pallas_call(
        _moe_kernel,
        grid_spec=pltpu.PrefetchScalarGridSpec(
            num_scalar_prefetch=1,
            grid=(E, NFB, NJB),
            in_specs=[
                pl.BlockSpec((S, D), lambda e, fb, jb, c: (0, 0)),
                pl.BlockSpec((S, 1), lambda e, fb, jb, c: (0, 0)),
                pl.BlockSpec((S, 1), lambda e, fb, jb, c: (0, 0)),
                pl.BlockSpec((S, 1), lambda e, fb, jb, c: (0, 0)),
                pl.BlockSpec((S, 1), lambda e, fb, jb, c: (0, 0)),
                pl.BlockSpec((1, 1, D), lambda e, fb, jb, c: (e, 0, 0)),
                pl.BlockSpec((1, 1, D), lambda e, fb, jb, c: (e, 0, 0)),
                pl.BlockSpec((1, FB, D), lambda e, fb, jb, c: (e, fb, 0)),
                pl.BlockSpec((1, 1, FB), lambda e, fb, jb, c: (e, 0, fb)),
                pl.BlockSpec((1, D, FB), lambda e, fb, jb, c: (e, 0, fb)),
                pl.BlockSpec((1, 1, D), lambda e, fb, jb, c: (e, 0, 0)),
            ],
            out_specs=pl.BlockSpec((S, D), lambda e, fb, jb, c: (0, 0)),
            scratch_shapes=[
                pltpu.VMEM((CAP, D), jnp.bfloat16),
                pltpu.VMEM((CAP, D), jnp.bfloat16),
            ],
        ),
        out_shape=jax.ShapeDtypeStruct((S, D), f32),
    )(cnt_i, z3b, slot1, slot2, wt1, wt2, eln_s[:, None, :],
      eln_b[:, None, :], eW1, eb1[:, None, :], eW2, eb2[:, None, :])

    out = pl.pallas_call(
        _final_kernel,
        grid=(S // TB,),
        in_specs=[
            pl.BlockSpec((TB, D), lambda t: (t, 0)),
            pl.BlockSpec((TB, D), lambda t: (t, 0)),
            pl.BlockSpec((1, D), lambda t: (0, 0)),
            pl.BlockSpec((1, D), lambda t: (0, 0)),
        ],
        out_specs=pl.BlockSpec((TB, D), lambda t: (t, 0)),
        out_shape=jax.ShapeDtypeStruct((S, D), f32),
    )(x2, comb, r1(n3s), r1(n3b))

    return out.reshape(1, S, D)


# speedup vs baseline: 3.3951x; 1.0092x over previous
import jax
import jax.numpy as jnp
from jax.experimental import pallas as pl
from jax.experimental.pallas import tpu as pltpu

S, D, H, E = 2048, 1024, 16, 8
DH = D // H
F = 4 * D
EPS = 1e-05

QB = 1024
TB = 512
FB = 1024
CAP = 2048
TBS = 512
NFB = F // FB
NJB = CAP // TBS
BIGSLOT = 1e6


def _lnorm(x, s, b):
    m = jnp.mean(x, axis=-1, keepdims=True)
    v = jnp.mean((x - m) ** 2, axis=-1, keepdims=True)
    return (x - m) * jax.lax.rsqrt(v + EPS) * s + b


def _qkv_kernel(x_ref, s_ref, b_ref, w_ref, bias_ref, o_ref):
    xln = _lnorm(x_ref[...], s_ref[...], b_ref[...])
    o_ref[...] = jax.lax.dot_general(
        xln, w_ref[...], (((1,), (1,)), ((), ())),
        preferred_element_type=jnp.float32) + bias_ref[...]


def _attn_kernel(q_ref, k_ref, v_ref, o_ref):
    outs = []
    for i in range(2):
        q = q_ref[:, i * DH:(i + 1) * DH]
        k = k_ref[:, i * DH:(i + 1) * DH]
        v = v_ref[:, i * DH:(i + 1) * DH]
        sc = jax.lax.dot_general(
            q, k, (((1,), (1,)), ((), ())),
            preferred_element_type=jnp.float32) * 0.125
        m = jnp.max(sc, axis=-1, keepdims=True)
        p = jnp.exp(sc - m)
        den = jnp.sum(p, axis=-1, keepdims=True)
        o = jnp.dot(p, v, preferred_element_type=jnp.float32)
        outs.append(o / den)
    o_ref[...] = jnp.concatenate(outs, axis=1)


def _post_kernel(x_ref, ao_ref, wo_ref, bo_ref, s2_ref, b2_ref, wr_ref,
                 br_ref, x2_ref, z3b_ref, s1_ref, s2o_ref, wt1_ref, wt2_ref,
                 cnt_ref, base_ref):
    tb = pl.program_id(0)

    @pl.when(tb == 0)
    def _():
        base_ref[...] = jnp.zeros_like(base_ref)

    attn = jax.lax.dot_general(
        ao_ref[...], wo_ref[...], (((1,), (1,)), ((), ())),
        preferred_element_type=jnp.float32) + bo_ref[...]
    x2 = x_ref[...] + attn
    x2_ref[...] = x2
    tok = _lnorm(x2, s2_ref[...], b2_ref[...])
    m = jnp.mean(tok, axis=-1, keepdims=True)
    v = jnp.mean((tok - m) ** 2, axis=-1, keepdims=True)
    z3b_ref[...] = ((tok - m) * jax.lax.rsqrt(v + EPS)).astype(jnp.bfloat16)
    lg = jax.lax.dot_general(
        tok, wr_ref[...], (((1,), (1,)), ((), ())),
        preferred_element_type=jnp.float32) + br_ref[...]

    iota = jax.lax.broadcasted_iota(jnp.int32, (TB, E), 1)
    m1 = jnp.max(lg, axis=-1, keepdims=True)
    i1 = jnp.min(jnp.where(lg == m1, iota, E), axis=-1, keepdims=True)
    lg2 = jnp.where(iota == i1, -jnp.inf, lg)
    m2 = jnp.max(lg2, axis=-1, keepdims=True)
    i2 = jnp.min(jnp.where(lg2 == m2, iota, E), axis=-1, keepdims=True)
    w1 = 1.0 / (1.0 + jnp.exp(m2 - m1))
    w2 = 1.0 - w1
    sel1 = iota == i1
    sel2 = iota == i2
    self_f = (sel1 | sel2).astype(jnp.float32)
    wt1_ref[...] = w1
    wt2_ref[...] = w2

    r_iota = jax.lax.broadcasted_iota(jnp.int32, (TB, TB), 0)
    c_iota = jax.lax.broadcasted_iota(jnp.int32, (TB, TB), 1)
    lt = (c_iota < r_iota).astype(jnp.float32)
    rank = jnp.floor(jnp.dot(lt, self_f,
                             preferred_element_type=jnp.float32) + 0.5)
    base = base_ref[...]
    e_cap = iota.astype(jnp.float32) * CAP
    gslot = e_cap + base + rank
    s1_ref[...] = jnp.sum(jnp.where(sel1, gslot, 0.0), axis=-1,
                          keepdims=True)
    s2o_ref[...] = jnp.sum(jnp.where(sel2, gslot, 0.0), axis=-1,
                           keepdims=True)
    base_ref[...] = base + jnp.sum(self_f, axis=0, keepdims=True)

    @pl.when(tb == pl.num_programs(0) - 1)
    def _():
        cnt_ref[...] = base_ref[...]


def _moe_kernel(cnt_ref, z3b_ref, s1_ref, s2_ref, wt1_ref, wt2_ref,
                es_ref, ebv_ref, w1_ref, b1_ref, w2_ref, b2_ref,
                out_ref, geo_ref, eo_ref):
    e = pl.program_id(0)
    fb = pl.program_id(1)
    jb = pl.program_id(2)

    @pl.when((e == 0) & (fb == 0) & (jb == 0))
    def _():
        out_ref[...] = jnp.zeros_like(out_ref)

    base = jb * TBS

    @pl.when(base < cnt_ref[e])
    def _work():
        off = (e * CAP + base).astype(jnp.float32)

        def _masks():
            lane = jax.lax.broadcasted_iota(
                jnp.int32, (S, TBS), 1).astype(jnp.float32)
            m1 = jnp.abs(s1_ref[...] - off - lane) < 0.5
            m2 = jnp.abs(s2_ref[...] - off - lane) < 0.5
            return m1, m2

        @pl.when(fb == 0)
        def _gather():
            m1, m2 = _masks()
            ptb = (m1 | m2).astype(jnp.bfloat16)
            g = jax.lax.dot_general(
                ptb, z3b_ref[...], (((0,), (0,)), ((), ())),
                preferred_element_type=jnp.float32)
            hl = g * es_ref[0] + ebv_ref[0]
            geo_ref[pl.ds(base, TBS), :] = hl.astype(jnp.bfloat16)

        hl = geo_ref[pl.ds(base, TBS), :]
        w1 = w1_ref[0].astype(jnp.bfloat16)
        hh = jax.lax.dot_general(
            hl, w1, (((1,), (1,)), ((), ())),
            preferred_element_type=jnp.float32) + b1_ref[0]
        gg = (hh * 0.5 * (1.0 + jax.lax.erf(hh * 0.7071067811865476))
              ).astype(jnp.bfloat16)
        w2 = w2_ref[0].astype(jnp.bfloat16)
        part = jax.lax.dot_general(
            gg, w2, (((1,), (1,)), ((), ())),
            preferred_element_type=jnp.float32)

        @pl.when(fb == 0)
        def _eo_init():
            eo_ref[pl.ds(base, TBS), :] = (part + b2_ref[0]).astype(
                jnp.bfloat16)

        @pl.when((fb > 0) & (fb < NFB - 1))
        def _eo_acc():
            eo_ref[pl.ds(base, TBS), :] = (
                eo_ref[pl.ds(base, TBS), :].astype(jnp.float32) + part
            ).astype(jnp.bfloat16)

        @pl.when(fb == NFB - 1)
        def _combine():
            eof = (eo_ref[pl.ds(base, TBS), :].astype(jnp.float32)
                   + part).astype(jnp.bfloat16)
            m1, m2 = _masks()
            wpt = (jnp.where(m1, wt1_ref[...], 0.0)
                   + jnp.where(m2, wt2_ref[...], 0.0)).astype(jnp.bfloat16)
            out_ref[...] += jax.lax.dot_general(
                wpt, eof, (((1,), (0,)), ((), ())),
                preferred_element_type=jnp.float32)


def _final_kernel(x2_ref, comb_ref, s3_ref, b3_ref, o_ref):
    o_ref[...] = _lnorm(x2_ref[...] + comb_ref[...], s3_ref[...], b3_ref[...])


def kernel(x, n1s, n1b, n2s, n2b, n3s, n3b, Wqkv, bqkv, Wo, bo, Wr, br,
           eln_s, eln_b, eW1, eb1, eW2, eb2):
    f32 = jnp.float32
    x2d = x.reshape(S, D)
    r1 = lambda a: a.reshape(1, -1)

    qkv = pl.pallas_call(
        _qkv_kernel,
        grid=(6,),
        in_specs=[
            pl.BlockSpec((S, D), lambda j: (0, 0)),
            pl.BlockSpec((1, D), lambda j: (0, 0)),
            pl.BlockSpec((1, D), lambda j: (0, 0)),
            pl.BlockSpec((512, D), lambda j: (j, 0)),
            pl.BlockSpec((1, 512), lambda j: (0, j)),
        ],
        out_specs=pl.BlockSpec((S, 512), lambda j: (0, j)),
        out_shape=jax.ShapeDtypeStruct((S, 3 * D), f32),
    )(x2d, r1(n1s), r1(n1b), Wqkv, r1(bqkv))

    ao = pl.pallas_call(
        _attn_kernel,
        grid=(H // 2, S // QB),
        in_specs=[
            pl.BlockSpec((QB, 2 * DH), lambda hp, qb: (qb, hp)),
            pl.BlockSpec((S, 2 * DH), lambda hp, qb: (0, H // 2 + hp)),
            pl.BlockSpec((S, 2 * DH), lambda hp, qb: (0, H + hp)),
        ],
        out_specs=pl.BlockSpec((QB, 2 * DH), lambda hp, qb: (qb, hp)),
        out_shape=jax.ShapeDtypeStruct((S, D), f32),
    )(qkv, qkv, qkv)

    x2, z3b, slot1, slot2, wt1, wt2, counts = pl.pallas_call(
        _post_kernel,
        grid=(S // TB,),
        in_specs=[
            pl.BlockSpec((TB, D), lambda t: (t, 0)),
            pl.BlockSpec((TB, D), lambda t: (t, 0)),
            pl.BlockSpec((D, D), lambda t: (0, 0)),
            pl.BlockSpec((1, D), lambda t: (0, 0)),
            pl.BlockSpec((1, D), lambda t: (0, 0)),
            pl.BlockSpec((1, D), lambda t: (0, 0)),
            pl.BlockSpec((E, D), lambda t: (0, 0)),
            pl.BlockSpec((1, E), lambda t: (0, 0)),
        ],
        out_specs=[
            pl.BlockSpec((TB, D), lambda t: (t, 0)),
            pl.BlockSpec((TB, D), lambda t: (t, 0)),
            pl.BlockSpec((TB, 1), lambda t: (t, 0)),
            pl.BlockSpec((TB, 1), lambda t: (t, 0)),
            pl.BlockSpec((TB, 1), lambda t: (t, 0)),
            pl.BlockSpec((TB, 1), lambda t: (t, 0)),
            pl.BlockSpec((1, E), lambda t: (0, 0)),
        ],
        out_shape=[
            jax.ShapeDtypeStruct((S, D), f32),
            jax.ShapeDtypeStruct((S, D), jnp.bfloat16),
            jax.ShapeDtypeStruct((S, 1), f32),
            jax.ShapeDtypeStruct((S, 1), f32),
            jax.ShapeDtypeStruct((S, 1), f32),
            jax.ShapeDtypeStruct((S, 1), f32),
            jax.ShapeDtypeStruct((1, E), f32),
        ],
        scratch_shapes=[pltpu.VMEM((1, E), f32)],
    )(x2d, ao, Wo, r1(bo), r1(n2s), r1(n2b), Wr, r1(br))

    cnt_i = counts.reshape(E).astype(jnp.int32)

    comb = pl.pallas_call(
        _moe_kernel,
        grid_spec=pltpu.PrefetchScalarGridSpec(
            num_scalar_prefetch=1,
            grid=(E, NFB, NJB),
            in_specs=[
                pl.BlockSpec((S, D), lambda e, fb, jb, c: (0, 0)),
                pl.BlockSpec((S, 1), lambda e, fb, jb, c: (0, 0)),
                pl.BlockSpec((S, 1), lambda e, fb, jb, c: (0, 0)),
                pl.BlockSpec((S, 1), lambda e, fb, jb, c: (0, 0)),
                pl.BlockSpec((S, 1), lambda e, fb, jb, c: (0, 0)),
                pl.BlockSpec((1, 1, D), lambda e, fb, jb, c: (e, 0, 0)),
                pl.BlockSpec((1, 1, D), lambda e, fb, jb, c: (e, 0, 0)),
                pl.BlockSpec((1, FB, D), lambda e, fb, jb, c: (e, fb, 0)),
                pl.BlockSpec((1, 1, FB), lambda e, fb, jb, c: (e, 0, fb)),
                pl.BlockSpec((1, D, FB), lambda e, fb, jb, c: (e, 0, fb)),
                pl.BlockSpec((1, 1, D), lambda e, fb, jb, c: (e, 0, 0)),
            ],
            out_specs=pl.BlockSpec((S, D), lambda e, fb, jb, c: (0, 0)),
            scratch_shapes=[
                pltpu.VMEM((CAP, D), jnp.bfloat16),
                pltpu.VMEM((CAP, D), jnp.bfloat16),
            ],
        ),
        out_shape=jax.ShapeDtypeStruct((S, D), f32),
    )(cnt_i, z3b, slot1, slot2, wt1, wt2, eln_s[:, None, :],
      eln_b[:, None, :], eW1, eb1[:, None, :], eW2, eb2[:, None, :])

    out = pl.pallas_call(
        _final_kernel,
        grid=(S // TB,),
        in_specs=[
            pl.BlockSpec((TB, D), lambda t: (t, 0)),
            pl.BlockSpec((TB, D), lambda t: (t, 0)),
            pl.BlockSpec((1, D), lambda t: (0, 0)),
            pl.BlockSpec((1, D), lambda t: (0, 0)),
        ],
        out_specs=pl.BlockSpec((TB, D), lambda t: (t, 0)),
        out_shape=jax.ShapeDtypeStruct((S, D), f32),
    )(x2, comb, r1(n3s), r1(n3b))

    return out.reshape(1, S, D)
